# Initial kernel scaffold; baseline (speedup 1.0000x reference)
import jax
import jax.numpy as jnp
from jax.experimental import pallas as pl


def kernel(inputs, router_w, w1, b1, w2, b2):
    raise NotImplementedError("write your pallas kernel here")


# speedup vs baseline: 4.2098x; 4.2098x over previous
import functools

import jax
import jax.numpy as jnp
from jax.experimental import pallas as pl
from jax.experimental.pallas import tpu as pltpu

NE = 64
DM = 1024
DFF = 2048
NT = 8192
K = 64
TBLK = 1024
FBLK = 512
NFB = DFF // FBLK


def _router_body(x_ref, rw_ref, logits_ref, probs_ref):
    x = x_ref[...]
    logits = jax.lax.dot_general(
        x, rw_ref[...], (((1,), (1,)), ((), ())),
        preferred_element_type=jnp.float32)
    logits_ref[...] = logits
    m = jnp.max(logits, axis=1, keepdims=True)
    e = jnp.exp(logits - m)
    probs_ref[...] = e / jnp.sum(e, axis=1, keepdims=True)


def _router(x, rw):
    return pl.pallas_call(
        _router_body,
        grid=(NT // TBLK,),
        in_specs=[
            pl.BlockSpec((TBLK, DM), lambda i: (i, 0)),
            pl.BlockSpec((NE, DM), lambda i: (0, 0)),
        ],
        out_specs=[
            pl.BlockSpec((TBLK, NE), lambda i: (i, 0)),
            pl.BlockSpec((TBLK, NE), lambda i: (i, 0)),
        ],
        out_shape=[
            jax.ShapeDtypeStruct((NT, NE), jnp.float32),
            jax.ShapeDtypeStruct((NT, NE), jnp.float32),
        ],
    )(x, rw)


def _topk_body(probs_ref, wt_ref, st_ref):
    p = probs_ref[...]
    row = jax.lax.broadcasted_iota(jnp.int32, (NT, NE), 0)

    def step(k, carry):
        last_v, last_i, W, S = carry
        elig = (p < last_v) | ((p == last_v) & (row > last_i))
        cand = jnp.where(elig, p, -1.0)
        v = jnp.max(cand, axis=0, keepdims=True)
        eq = cand == v
        idx = jnp.min(jnp.where(eq, row, NT), axis=0, keepdims=True)
        krow = jax.lax.broadcasted_iota(jnp.int32, (K, NE), 0) == k
        W = jnp.where(krow, v, W)
        S = jnp.where(krow, idx, S)
        return v, idx, W, S

    init = (jnp.full((1, NE), jnp.inf, jnp.float32),
            jnp.full((1, NE), -1, jnp.int32),
            jnp.zeros((K, NE), jnp.float32),
            jnp.zeros((K, NE), jnp.int32))
    _, _, W, S = jax.lax.fori_loop(0, K, step, init)
    wt_ref[...] = W
    st_ref[...] = S


def _topk(probs):
    return pl.pallas_call(
        _topk_body,
        out_shape=[
            jax.ShapeDtypeStruct((K, NE), jnp.float32),
            jax.ShapeDtypeStruct((K, NE), jnp.int32),
        ],
    )(probs)


def _mlp_body(sel_ref, x_ref, w1_ref, b1_ref, w2_ref, b2_ref, wt_ref,
              y_ref, xt_ref, acc_ref):
    e = pl.program_id(0)
    c = pl.program_id(1)

    @pl.when(c == 0)
    def _():
        def gather(i, _):
            t = sel_ref[e * K + i]
            xt_ref[pl.ds(i, 1), :] = x_ref[pl.ds(t, 1), :]
            return 0
        jax.lax.fori_loop(0, K, gather, 0)
        acc_ref[...] = jnp.zeros_like(acc_ref)

    h = jnp.dot(xt_ref[...], w1_ref[0], preferred_element_type=jnp.float32)
    h = h + b1_ref[0]
    h = 0.5 * h * (1.0 + jax.lax.erf(h * 0.7071067811865476))
    acc_ref[...] += jnp.dot(h, w2_ref[0], preferred_element_type=jnp.float32)

    @pl.when(c == NFB - 1)
    def _():
        y_ref[0] = (acc_ref[...] + b2_ref[0]) * wt_ref[0]


def _mlp(sel_flat, x, w1, b1, w2, b2, wt_ecol):
    grid_spec = pltpu.PrefetchScalarGridSpec(
        num_scalar_prefetch=1,
        grid=(NE, NFB),
        in_specs=[
            pl.BlockSpec((NT, DM), lambda e, c, sel: (0, 0)),
            pl.BlockSpec((1, DM, FBLK), lambda e, c, sel: (e, 0, c)),
            pl.BlockSpec((1, 1, FBLK), lambda e, c, sel: (e, 0, c)),
            pl.BlockSpec((1, FBLK, DM), lambda e, c, sel: (e, c, 0)),
            pl.BlockSpec((1, 1, DM), lambda e, c, sel: (e, 0, 0)),
            pl.BlockSpec((1, K, 1), lambda e, c, sel: (e, 0, 0)),
        ],
        out_specs=pl.BlockSpec((1, K, DM), lambda e, c, sel: (e, 0, 0)),
        scratch_shapes=[
            pltpu.VMEM((K, DM), jnp.float32),
            pltpu.VMEM((K, DM), jnp.float32),
        ],
    )
    return pl.pallas_call(
        _mlp_body,
        grid_spec=grid_spec,
        out_shape=jax.ShapeDtypeStruct((NE, K, DM), jnp.float32),
    )(sel_flat, x, w1.reshape(NE, DM, DFF), b1.reshape(NE, 1, DFF),
      w2.reshape(NE, DFF, DM), b2.reshape(NE, 1, DM), wt_ecol)


def _combine_body(sel_ref, y_ref, res_ref):
    e = pl.program_id(0)

    @pl.when(e == 0)
    def _():
        res_ref[...] = jnp.zeros_like(res_ref)

    def scatter(i, _):
        t = sel_ref[e * K + i]
        res_ref[pl.ds(t, 1), :] += y_ref[0, pl.ds(i, 1), :]
        return 0
    jax.lax.fori_loop(0, K, scatter, 0)


def _combine(sel_flat, y):
    grid_spec = pltpu.PrefetchScalarGridSpec(
        num_scalar_prefetch=1,
        grid=(NE,),
        in_specs=[pl.BlockSpec((1, K, DM), lambda e, sel: (e, 0, 0))],
        out_specs=pl.BlockSpec((NT, DM), lambda e, sel: (0, 0)),
    )
    return pl.pallas_call(
        _combine_body,
        grid_spec=grid_spec,
        out_shape=jax.ShapeDtypeStruct((NT, DM), jnp.float32),
    )(sel_flat, y)


def kernel(inputs, router_w, w1, b1, w2, b2):
    x = inputs.reshape(NT, DM)
    logits, probs = _router(x, router_w)
    wt, st = _topk(probs)
    selected = st.T
    sel_flat = selected.reshape(-1)
    y = _mlp(sel_flat, x, w1, b1, w2, b2, wt.T.reshape(NE, K, 1))
    results = _combine(sel_flat, y)
    return results.reshape(inputs.shape), logits, selected


# speedup vs baseline: 4.4233x; 1.0507x over previous
import functools

import jax
import jax.numpy as jnp
from jax import lax
from jax.experimental import pallas as pl
from jax.experimental.pallas import tpu as pltpu
from jax.experimental.pallas import tpu_sc as plsc

NE = 64
DM = 1024
DFF = 2048
NT = 8192
K = 64
TBLK = 1024
FBLK = 512
NFB = DFF // FBLK

NC = 2
NS = 16
NW = NC * NS
RPW = NE * K // NW
RCHUNK = 64


def _router_body(x_ref, rw_ref, logits_ref, probs_ref):
    x = x_ref[...]
    logits = jax.lax.dot_general(
        x, rw_ref[...], (((1,), (1,)), ((), ())),
        preferred_element_type=jnp.float32)
    logits_ref[...] = logits
    m = jnp.max(logits, axis=1, keepdims=True)
    e = jnp.exp(logits - m)
    probs_ref[...] = e / jnp.sum(e, axis=1, keepdims=True)


def _router(x, rw):
    return pl.pallas_call(
        _router_body,
        grid=(NT // TBLK,),
        in_specs=[
            pl.BlockSpec((TBLK, DM), lambda i: (i, 0)),
            pl.BlockSpec((NE, DM), lambda i: (0, 0)),
        ],
        out_specs=[
            pl.BlockSpec((TBLK, NE), lambda i: (i, 0)),
            pl.BlockSpec((TBLK, NE), lambda i: (i, 0)),
        ],
        out_shape=[
            jax.ShapeDtypeStruct((NT, NE), jnp.float32),
            jax.ShapeDtypeStruct((NT, NE), jnp.float32),
        ],
    )(x, rw)


def _topk_body(probs_ref, wt_ref, st_ref):
    p = probs_ref[...]
    row = jax.lax.broadcasted_iota(jnp.int32, (NT, NE), 0)

    def step(k, carry):
        last_v, last_i, W, S = carry
        elig = (p < last_v) | ((p == last_v) & (row > last_i))
        cand = jnp.where(elig, p, -1.0)
        v = jnp.max(cand, axis=0, keepdims=True)
        eq = cand == v
        idx = jnp.min(jnp.where(eq, row, NT), axis=0, keepdims=True)
        krow = jax.lax.broadcasted_iota(jnp.int32, (K, NE), 0) == k
        W = jnp.where(krow, v, W)
        S = jnp.where(krow, idx, S)
        return v, idx, W, S

    init = (jnp.full((1, NE), jnp.inf, jnp.float32),
            jnp.full((1, NE), -1, jnp.int32),
            jnp.zeros((K, NE), jnp.float32),
            jnp.zeros((K, NE), jnp.int32))
    _, _, W, S = jax.lax.fori_loop(0, K, step, init)
    wt_ref[...] = W
    st_ref[...] = S


def _topk(probs):
    return pl.pallas_call(
        _topk_body,
        out_shape=[
            jax.ShapeDtypeStruct((K, NE), jnp.float32),
            jax.ShapeDtypeStruct((K, NE), jnp.int32),
        ],
    )(probs)


def _sc_gather_body(x_hbm, sel_hbm, xg_hbm, idx_v, rows_v, sem):
    wid = lax.axis_index("s") * NC + lax.axis_index("c")
    base = wid * RPW
    pltpu.sync_copy(sel_hbm.at[pl.ds(base, RPW)], idx_v)
    for chunk in range(RPW // RCHUNK):
        pltpu.async_copy(
            x_hbm.at[idx_v.at[pl.ds(chunk * RCHUNK, RCHUNK)]],
            rows_v, sem).wait()
        pltpu.sync_copy(
            rows_v, xg_hbm.at[pl.ds(base + chunk * RCHUNK, RCHUNK)])


def _sc_gather(x, sel_flat):
    mesh = plsc.VectorSubcoreMesh(core_axis_name="c", subcore_axis_name="s")
    f = pl.kernel(
        _sc_gather_body, mesh=mesh,
        out_type=jax.ShapeDtypeStruct((NE * K, DM), jnp.float32),
        scratch_types=[
            pltpu.VMEM((RPW,), jnp.int32),
            pltpu.VMEM((RCHUNK, DM), jnp.float32),
            pltpu.SemaphoreType.DMA,
        ],
    )
    return f(x, sel_flat)


def _mlp_body(sel_ref, xg_ref, w1_ref, b1_ref, w2_ref, b2_ref, wt_ref,
              res_ref, acc_ref):
    e = pl.program_id(0)
    c = pl.program_id(1)

    @pl.when((e == 0) & (c == 0))
    def _():
        res_ref[...] = jnp.zeros_like(res_ref)

    @pl.when(c == 0)
    def _():
        acc_ref[...] = jnp.zeros_like(acc_ref)

    h = jnp.dot(xg_ref[0].astype(jnp.bfloat16),
                w1_ref[0].astype(jnp.bfloat16),
                preferred_element_type=jnp.float32)
    h = h + b1_ref[0]
    h = 0.5 * h * (1.0 + jax.lax.erf(h * 0.7071067811865476))
    acc_ref[...] += jnp.dot(h.astype(jnp.bfloat16),
                            w2_ref[0].astype(jnp.bfloat16),
                            preferred_element_type=jnp.float32)

    @pl.when(c == NFB - 1)
    def _():
        acc_ref[...] = (acc_ref[...] + b2_ref[0]) * wt_ref[0]

        def scatter(i, _):
            t = sel_ref[e * K + i]
            res_ref[pl.ds(t, 1), :] += acc_ref[pl.ds(i, 1), :]
            return 0
        jax.lax.fori_loop(0, K, scatter, 0)


def _mlp(sel_flat, xg, w1, b1, w2, b2, wt_ecol):
    grid_spec = pltpu.PrefetchScalarGridSpec(
        num_scalar_prefetch=1,
        grid=(NE, NFB),
        in_specs=[
            pl.BlockSpec((1, K, DM), lambda e, c, sel: (e, 0, 0)),
            pl.BlockSpec((1, DM, FBLK), lambda e, c, sel: (e, 0, c)),
            pl.BlockSpec((1, 1, FBLK), lambda e, c, sel: (e, 0, c)),
            pl.BlockSpec((1, FBLK, DM), lambda e, c, sel: (e, c, 0)),
            pl.BlockSpec((1, 1, DM), lambda e, c, sel: (e, 0, 0)),
            pl.BlockSpec((1, K, 1), lambda e, c, sel: (e, 0, 0)),
        ],
        out_specs=pl.BlockSpec((NT, DM), lambda e, c, sel: (0, 0)),
        scratch_shapes=[
            pltpu.VMEM((K, DM), jnp.float32),
        ],
    )
    return pl.pallas_call(
        _mlp_body,
        grid_spec=grid_spec,
        out_shape=jax.ShapeDtypeStruct((NT, DM), jnp.float32),
    )(sel_flat, xg.reshape(NE, K, DM), w1.reshape(NE, DM, DFF),
      b1.reshape(NE, 1, DFF), w2.reshape(NE, DFF, DM),
      b2.reshape(NE, 1, DM), wt_ecol)


def kernel(inputs, router_w, w1, b1, w2, b2):
    x = inputs.reshape(NT, DM)
    logits, probs = _router(x, router_w)
    wt, st = _topk(probs)
    selected = st.T
    sel_flat = selected.reshape(-1)
    xg = _sc_gather(x, sel_flat)
    results = _mlp(sel_flat, xg, w1, b1, w2, b2, wt.T.reshape(NE, K, 1))
    return results.reshape(inputs.shape), logits, selected
